# Initial kernel scaffold; baseline (speedup 1.0000x reference)
import jax
import jax.numpy as jnp
from jax.experimental import pallas as pl


def kernel(node_features, edge_index, edge_type, proj_W, proj_b, edge_W0, edge_b0, gru_Wih0, gru_Whh0, gru_bih0, gru_bhh0, edge_W1, edge_b1, gru_Wih1, gru_Whh1, gru_bih1, gru_bhh1, out_W, out_b):
    raise NotImplementedError("write your pallas kernel here")


# speedup vs baseline: 27.9488x; 27.9488x over previous
import functools

import jax
import jax.numpy as jnp
from jax import lax
from jax.experimental import pallas as pl
from jax.experimental.pallas import tpu as pltpu
from jax.experimental.pallas import tpu_sc as plsc

N = 10000
E = 320000
D = 128
T = 8

NC = 2
NS = 16
NW = NC * NS
EPW = E // NW
K = 40
SC_CH = 50
SB = EPW // (K * SC_CH)
NPAD = 10112
RPS = NPAD // NS

NB = 400
GRID_N = N // NB


def _linear_body(relu, x_ref, w_ref, b_ref, o_ref):
    acc = lax.dot_general(x_ref[...], w_ref[...], (((1,), (1,)), ((), ())),
                          preferred_element_type=jnp.float32)
    acc = acc + b_ref[...]
    if relu:
        acc = jnp.maximum(acc, 0.0)
    o_ref[...] = acc


def _linear(x, w, b, relu):
    n, d = x.shape
    o = w.shape[0]
    return pl.pallas_call(
        functools.partial(_linear_body, relu),
        grid=(n // NB,),
        in_specs=[
            pl.BlockSpec((NB, d), lambda i: (i, 0)),
            pl.BlockSpec((o, d), lambda i: (0, 0)),
            pl.BlockSpec((1, o), lambda i: (0, 0)),
        ],
        out_specs=pl.BlockSpec((NB, o), lambda i: (i, 0)),
        out_shape=jax.ShapeDtypeStruct((n, o), jnp.float32),
    )(x, w, b.reshape(1, o))


def _transform_body(h_ref, w_ref, b_ref, o_ref):
    h = h_ref[...]
    for t in range(T):
        acc = lax.dot_general(h, w_ref[t], (((1,), (1,)), ((), ())),
                              preferred_element_type=jnp.float32)
        o_ref[t] = acc + b_ref[t]


def _transform(h, eW, eb):
    out = pl.pallas_call(
        _transform_body,
        grid=(GRID_N,),
        in_specs=[
            pl.BlockSpec((NB, D), lambda i: (i, 0)),
            pl.BlockSpec((T, D, D), lambda i: (0, 0, 0)),
            pl.BlockSpec((T, D), lambda i: (0, 0)),
        ],
        out_specs=pl.BlockSpec((T, NB, D), lambda i: (0, i, 0)),
        out_shape=jax.ShapeDtypeStruct((T, N, D), jnp.float32),
    )(h, eW, eb)
    return out.reshape(T * N, D)


def _gru_body(agg_ref, h_ref, wih_ref, whh_ref, bih_ref, bhh_ref, o_ref):
    a = agg_ref[0] + agg_ref[1]
    h = h_ref[...]
    gi = lax.dot_general(a, wih_ref[...], (((1,), (1,)), ((), ())),
                         preferred_element_type=jnp.float32) + bih_ref[...]
    gh = lax.dot_general(h, whh_ref[...], (((1,), (1,)), ((), ())),
                         preferred_element_type=jnp.float32) + bhh_ref[...]
    r = jax.nn.sigmoid(gi[:, :D] + gh[:, :D])
    z = jax.nn.sigmoid(gi[:, D:2 * D] + gh[:, D:2 * D])
    n = jnp.tanh(gi[:, 2 * D:] + r * gh[:, 2 * D:])
    o_ref[...] = jnp.maximum((1.0 - z) * n + z * h, 0.0)


def _gru(agg2, h, wih, whh, bih, bhh):
    return pl.pallas_call(
        _gru_body,
        grid=(GRID_N,),
        in_specs=[
            pl.BlockSpec((NC, NB, D), lambda i: (0, i, 0)),
            pl.BlockSpec((NB, D), lambda i: (i, 0)),
            pl.BlockSpec((3 * D, D), lambda i: (0, 0)),
            pl.BlockSpec((3 * D, D), lambda i: (0, 0)),
            pl.BlockSpec((1, 3 * D), lambda i: (0, 0)),
            pl.BlockSpec((1, 3 * D), lambda i: (0, 0)),
        ],
        out_specs=pl.BlockSpec((NB, D), lambda i: (i, 0)),
        out_shape=jax.ShapeDtypeStruct((N, D), jnp.float32),
    )(agg2, h, wih, whh, bih.reshape(1, 3 * D), bhh.reshape(1, 3 * D))


def _sc_scatter(trans_flat, gidx, didx):
    mesh = plsc.VectorSubcoreMesh(core_axis_name="c", subcore_axis_name="s")

    @functools.partial(
        pl.kernel,
        out_type=jax.ShapeDtypeStruct((NC, NPAD, D), jnp.float32),
        mesh=mesh,
        scratch_types=[
            pltpu.VMEM((2, SC_CH, K), jnp.int32),
            pltpu.VMEM((2, SC_CH, K), jnp.int32),
            pltpu.VMEM((K, D), jnp.float32),
            pltpu.VMEM((K, D), jnp.float32),
            pltpu.VMEM_SHARED((NPAD, D), jnp.float32),
            pltpu.SemaphoreType.DMA,
            pltpu.SemaphoreType.DMA,
            pltpu.SemaphoreType.DMA,
            pltpu.SemaphoreType.DMA,
        ],
    )
    def k(trans_hbm, gidx_hbm, didx_hbm, out_hbm,
          gidx_v, didx_v, rows0, rows1, agg_sh, sem0, sem1, semg, semd):
        cid = lax.axis_index("c")
        sid = lax.axis_index("s")
        wid = sid * NC + cid

        @pl.loop(0, K)
        def _(r):
            for c16 in range(D // 16):
                rows0[r, pl.ds(c16 * 16, 16)] = jnp.zeros((16,), jnp.float32)
        for z in range(RPS // K):
            pltpu.sync_copy(rows0, agg_sh.at[pl.ds(sid * RPS + z * K, K)])
        pltpu.sync_copy(rows0.at[pl.ds(0, RPS % K)],
                        agg_sh.at[pl.ds(sid * RPS + (RPS // K) * K, RPS % K)])
        plsc.subcore_barrier()

        def wait_rows(buf, sem):
            pltpu.make_async_copy(trans_hbm.at[pl.ds(0, K)], buf, sem).wait()

        pltpu.sync_copy(gidx_hbm.at[wid].at[0], gidx_v.at[0])
        pltpu.sync_copy(didx_hbm.at[wid].at[0], didx_v.at[0])
        idx_cp = None
        for sb in range(SB):
            ib = sb % 2
            if idx_cp is not None:
                for cp in idx_cp:
                    cp.wait()
            if sb + 1 < SB:
                idx_cp = (
                    pltpu.async_copy(gidx_hbm.at[wid].at[sb + 1],
                                     gidx_v.at[1 - ib], semg),
                    pltpu.async_copy(didx_hbm.at[wid].at[sb + 1],
                                     didx_v.at[1 - ib], semd),
                )
            gv, dv = gidx_v.at[ib], didx_v.at[ib]
            pltpu.async_copy(trans_hbm.at[gv.at[0]], rows0, sem0)

            @pl.loop(0, SC_CH // 2 - 1)
            def _(p):
                c0 = 2 * p
                pltpu.async_copy(trans_hbm.at[gv.at[c0 + 1]], rows1, sem1)
                wait_rows(rows0, sem0)
                pltpu.sync_copy(rows0, agg_sh.at[dv.at[c0]], add=True)
                pltpu.async_copy(trans_hbm.at[gv.at[c0 + 2]], rows0, sem0)
                wait_rows(rows1, sem1)
                pltpu.sync_copy(rows1, agg_sh.at[dv.at[c0 + 1]], add=True)

            pltpu.async_copy(trans_hbm.at[gv.at[SC_CH - 1]], rows1, sem1)
            wait_rows(rows0, sem0)
            pltpu.sync_copy(rows0, agg_sh.at[dv.at[SC_CH - 2]], add=True)
            wait_rows(rows1, sem1)
            pltpu.sync_copy(rows1, agg_sh.at[dv.at[SC_CH - 1]], add=True)

        plsc.subcore_barrier()
        pltpu.sync_copy(agg_sh.at[pl.ds(sid * RPS, RPS)],
                        out_hbm.at[cid].at[pl.ds(sid * RPS, RPS)])

    return k(trans_flat, gidx, didx)


def kernel(node_features, edge_index, edge_type, proj_W, proj_b,
           edge_W0, edge_b0, gru_Wih0, gru_Whh0, gru_bih0, gru_bhh0,
           edge_W1, edge_b1, gru_Wih1, gru_Whh1, gru_bih1, gru_bhh1,
           out_W, out_b):
    src = edge_index[0].astype(jnp.int32)
    dst = edge_index[1].astype(jnp.int32)
    gidx = (edge_type.astype(jnp.int32) * N + src).reshape(NW, SB, SC_CH, K)
    didx = dst.reshape(NW, SB, SC_CH, K)

    h = _linear(node_features, proj_W, proj_b, relu=True)
    layers = [
        (edge_W0, edge_b0, gru_Wih0, gru_Whh0, gru_bih0, gru_bhh0),
        (edge_W1, edge_b1, gru_Wih1, gru_Whh1, gru_bih1, gru_bhh1),
    ]
    for _round in range(2):
        for (eW, eb, Wih, Whh, bih, bhh) in layers:
            trans = _transform(h, eW, eb)
            agg2 = _sc_scatter(trans, gidx, didx)
            h = _gru(agg2, h, Wih, Whh, bih, bhh)
    return _linear(h, out_W, out_b, relu=False)


# speedup vs baseline: 33.8197x; 1.2101x over previous
import functools

import jax
import jax.numpy as jnp
from jax import lax
from jax.experimental import pallas as pl
from jax.experimental.pallas import tpu as pltpu
from jax.experimental.pallas import tpu_sc as plsc

N = 10000
E = 320000
D = 128
T = 8

NC = 2
NS = 16
NW = NC * NS
EPW = E // NW
K = 40
SC_CH = 50
SB = EPW // (K * SC_CH)
NBUF = 4
NG = SC_CH // NBUF
NTAIL = SC_CH - NG * NBUF
NPAD = 10112
RPS = NPAD // NS

NB = 400
GRID_N = N // NB


def _linear_body(relu, x_ref, w_ref, b_ref, o_ref):
    acc = lax.dot_general(x_ref[...], w_ref[...], (((1,), (1,)), ((), ())),
                          preferred_element_type=jnp.float32)
    acc = acc + b_ref[...]
    if relu:
        acc = jnp.maximum(acc, 0.0)
    o_ref[...] = acc


def _linear(x, w, b, relu):
    n, d = x.shape
    o = w.shape[0]
    return pl.pallas_call(
        functools.partial(_linear_body, relu),
        grid=(n // NB,),
        in_specs=[
            pl.BlockSpec((NB, d), lambda i: (i, 0)),
            pl.BlockSpec((o, d), lambda i: (0, 0)),
            pl.BlockSpec((1, o), lambda i: (0, 0)),
        ],
        out_specs=pl.BlockSpec((NB, o), lambda i: (i, 0)),
        out_shape=jax.ShapeDtypeStruct((n, o), jnp.float32),
    )(x, w, b.reshape(1, o))


def _transform_body(h_ref, w_ref, b_ref, o_ref):
    h = h_ref[...]
    for t in range(T):
        acc = lax.dot_general(h, w_ref[t], (((1,), (1,)), ((), ())),
                              preferred_element_type=jnp.float32)
        o_ref[t] = acc + b_ref[t]


def _transform(h, eW, eb):
    out = pl.pallas_call(
        _transform_body,
        grid=(GRID_N,),
        in_specs=[
            pl.BlockSpec((NB, D), lambda i: (i, 0)),
            pl.BlockSpec((T, D, D), lambda i: (0, 0, 0)),
            pl.BlockSpec((T, D), lambda i: (0, 0)),
        ],
        out_specs=pl.BlockSpec((T, NB, D), lambda i: (0, i, 0)),
        out_shape=jax.ShapeDtypeStruct((T, N, D), jnp.float32),
    )(h, eW, eb)
    return out.reshape(T * N, D)


def _gru_body(agg_ref, h_ref, wih_ref, whh_ref, bih_ref, bhh_ref, o_ref):
    a = agg_ref[0] + agg_ref[1]
    h = h_ref[...]
    gi = lax.dot_general(a, wih_ref[...], (((1,), (1,)), ((), ())),
                         preferred_element_type=jnp.float32) + bih_ref[...]
    gh = lax.dot_general(h, whh_ref[...], (((1,), (1,)), ((), ())),
                         preferred_element_type=jnp.float32) + bhh_ref[...]
    r = jax.nn.sigmoid(gi[:, :D] + gh[:, :D])
    z = jax.nn.sigmoid(gi[:, D:2 * D] + gh[:, D:2 * D])
    n = jnp.tanh(gi[:, 2 * D:] + r * gh[:, 2 * D:])
    o_ref[...] = jnp.maximum((1.0 - z) * n + z * h, 0.0)


def _gru(agg2, h, wih, whh, bih, bhh):
    return pl.pallas_call(
        _gru_body,
        grid=(GRID_N,),
        in_specs=[
            pl.BlockSpec((NC, NB, D), lambda i: (0, i, 0)),
            pl.BlockSpec((NB, D), lambda i: (i, 0)),
            pl.BlockSpec((3 * D, D), lambda i: (0, 0)),
            pl.BlockSpec((3 * D, D), lambda i: (0, 0)),
            pl.BlockSpec((1, 3 * D), lambda i: (0, 0)),
            pl.BlockSpec((1, 3 * D), lambda i: (0, 0)),
        ],
        out_specs=pl.BlockSpec((NB, D), lambda i: (i, 0)),
        out_shape=jax.ShapeDtypeStruct((N, D), jnp.float32),
    )(agg2, h, wih, whh, bih.reshape(1, 3 * D), bhh.reshape(1, 3 * D))


def _sc_scatter(trans_flat, gidx, didx):
    mesh = plsc.VectorSubcoreMesh(core_axis_name="c", subcore_axis_name="s")

    @functools.partial(
        pl.kernel,
        out_type=jax.ShapeDtypeStruct((NC, NPAD, D), jnp.float32),
        mesh=mesh,
        scratch_types=[
            pltpu.VMEM((2, SC_CH, K), jnp.int32),
            pltpu.VMEM((2, SC_CH, K), jnp.int32),
            pltpu.VMEM((NBUF, K, D), jnp.float32),
            pltpu.VMEM_SHARED((NPAD, D), jnp.float32),
        ] + [pltpu.SemaphoreType.DMA] * (2 * NBUF + 2),
    )
    def k(trans_hbm, gidx_hbm, didx_hbm, out_hbm,
          gidx_v, didx_v, rows, agg_sh, *sems):
        gsems = sems[:NBUF]
        ssems = sems[NBUF:2 * NBUF]
        semg, semd = sems[2 * NBUF], sems[2 * NBUF + 1]
        cid = lax.axis_index("c")
        sid = lax.axis_index("s")
        wid = sid * NC + cid

        @pl.loop(0, K)
        def _(r):
            for c16 in range(D // 16):
                rows[0, r, pl.ds(c16 * 16, 16)] = jnp.zeros((16,), jnp.float32)
        for z in range(RPS // K):
            pltpu.sync_copy(rows.at[0], agg_sh.at[pl.ds(sid * RPS + z * K, K)])
        pltpu.sync_copy(rows.at[0].at[pl.ds(0, RPS % K)],
                        agg_sh.at[pl.ds(sid * RPS + (RPS // K) * K, RPS % K)])
        plsc.subcore_barrier()

        def start_gather(gv, c, b):
            pltpu.async_copy(trans_hbm.at[gv.at[c]], rows.at[b], gsems[b])

        def wait_gather(b):
            pltpu.make_async_copy(trans_hbm.at[pl.ds(0, K)], rows.at[b],
                                  gsems[b]).wait()

        def start_scatter(dv, c, b):
            pltpu.async_copy(rows.at[b], agg_sh.at[dv.at[c]], ssems[b],
                             add=True)

        def wait_scatter(b):
            pltpu.make_async_copy(rows.at[b], agg_sh.at[pl.ds(0, K)],
                                  ssems[b]).wait()

        pltpu.sync_copy(gidx_hbm.at[wid].at[0], gidx_v.at[0])
        pltpu.sync_copy(didx_hbm.at[wid].at[0], didx_v.at[0])
        idx_cp = None
        for sb in range(SB):
            ib = sb % 2
            if idx_cp is not None:
                for cp in idx_cp:
                    cp.wait()
            if sb + 1 < SB:
                idx_cp = (
                    pltpu.async_copy(gidx_hbm.at[wid].at[sb + 1],
                                     gidx_v.at[1 - ib], semg),
                    pltpu.async_copy(didx_hbm.at[wid].at[sb + 1],
                                     didx_v.at[1 - ib], semd),
                )
            gv, dv = gidx_v.at[ib], didx_v.at[ib]
            for b in range(NBUF):
                start_gather(gv, b, b)

            @pl.loop(1, NG)
            def _(i):
                c = NBUF * i
                for b in range(NBUF):
                    wait_gather(b)
                    start_scatter(dv, c - NBUF + b, b)
                for b in range(NBUF):
                    wait_scatter(b)
                    start_gather(gv, c + b, b)

            for b in range(NBUF):
                wait_gather(b)
                start_scatter(dv, (NG - 1) * NBUF + b, b)
            for t in range(NTAIL):
                wait_scatter(t)
                start_gather(gv, NG * NBUF + t, t)
            for t in range(NTAIL):
                wait_gather(t)
                start_scatter(dv, NG * NBUF + t, t)
            for b in range(NBUF):
                wait_scatter(b)

        plsc.subcore_barrier()
        pltpu.sync_copy(agg_sh.at[pl.ds(sid * RPS, RPS)],
                        out_hbm.at[cid].at[pl.ds(sid * RPS, RPS)])

    return k(trans_flat, gidx, didx)


def kernel(node_features, edge_index, edge_type, proj_W, proj_b,
           edge_W0, edge_b0, gru_Wih0, gru_Whh0, gru_bih0, gru_bhh0,
           edge_W1, edge_b1, gru_Wih1, gru_Whh1, gru_bih1, gru_bhh1,
           out_W, out_b):
    src = edge_index[0].astype(jnp.int32)
    dst = edge_index[1].astype(jnp.int32)
    gidx = (edge_type.astype(jnp.int32) * N + src).reshape(NW, SB, SC_CH, K)
    didx = dst.reshape(NW, SB, SC_CH, K)

    h = _linear(node_features, proj_W, proj_b, relu=True)
    layers = [
        (edge_W0, edge_b0, gru_Wih0, gru_Whh0, gru_bih0, gru_bhh0),
        (edge_W1, edge_b1, gru_Wih1, gru_Whh1, gru_bih1, gru_bhh1),
    ]
    for _round in range(2):
        for (eW, eb, Wih, Whh, bih, bhh) in layers:
            trans = _transform(h, eW, eb)
            agg2 = _sc_scatter(trans, gidx, didx)
            h = _gru(agg2, h, Wih, Whh, bih, bhh)
    return _linear(h, out_W, out_b, relu=False)


# speedup vs baseline: 37.5519x; 1.1104x over previous
import functools

import jax
import jax.numpy as jnp
from jax import lax
from jax.experimental import pallas as pl
from jax.experimental.pallas import tpu as pltpu
from jax.experimental.pallas import tpu_sc as plsc

N = 10000
E = 320000
D = 128
T = 8

NC = 2
NS = 16
NW = NC * NS
EPW = E // NW
K = 40
SC_CH = 50
SB = EPW // (K * SC_CH)
NBUF = 4
NG = SC_CH // NBUF
NTAIL = SC_CH - NG * NBUF
NPAD = 10112
RPS = NPAD // NS

NB = 400
GRID_N = N // NB


def _gru_block(agg_ref, h_ref, wih_ref, whh_ref, bih_ref, bhh_ref):
    a = agg_ref[0] + agg_ref[1]
    h = h_ref[...]
    gi = lax.dot_general(a, wih_ref[...], (((1,), (1,)), ((), ())),
                         preferred_element_type=jnp.float32) + bih_ref[...]
    gh = lax.dot_general(h, whh_ref[...], (((1,), (1,)), ((), ())),
                         preferred_element_type=jnp.float32) + bhh_ref[...]
    r = jax.nn.sigmoid(gi[:, :D] + gh[:, :D])
    z = jax.nn.sigmoid(gi[:, D:2 * D] + gh[:, D:2 * D])
    n = jnp.tanh(gi[:, 2 * D:] + r * gh[:, 2 * D:])
    return jnp.maximum((1.0 - z) * n + z * h, 0.0)


def _trans_block(h, ew_ref, eb_ref, tr_ref):
    for t in range(T):
        acc = lax.dot_general(h, ew_ref[t], (((1,), (1,)), ((), ())),
                              preferred_element_type=jnp.float32)
        tr_ref[t] = acc + eb_ref[t]


_GRU_SPECS = [
    pl.BlockSpec((NC, NB, D), lambda i: (0, i, 0)),
    pl.BlockSpec((NB, D), lambda i: (i, 0)),
    pl.BlockSpec((3 * D, D), lambda i: (0, 0)),
    pl.BlockSpec((3 * D, D), lambda i: (0, 0)),
    pl.BlockSpec((1, 3 * D), lambda i: (0, 0)),
    pl.BlockSpec((1, 3 * D), lambda i: (0, 0)),
]


def _proj_trans_body(x_ref, pw_ref, pb_ref, ew_ref, eb_ref, h_ref, tr_ref):
    acc = lax.dot_general(x_ref[...], pw_ref[...], (((1,), (1,)), ((), ())),
                          preferred_element_type=jnp.float32)
    h = jnp.maximum(acc + pb_ref[...], 0.0)
    h_ref[...] = h
    _trans_block(h, ew_ref, eb_ref, tr_ref)


def _proj_trans(x, pw, pb, ew, eb):
    h, tr = pl.pallas_call(
        _proj_trans_body,
        grid=(GRID_N,),
        in_specs=[
            pl.BlockSpec((NB, D), lambda i: (i, 0)),
            pl.BlockSpec((D, D), lambda i: (0, 0)),
            pl.BlockSpec((1, D), lambda i: (0, 0)),
            pl.BlockSpec((T, D, D), lambda i: (0, 0, 0)),
            pl.BlockSpec((T, D), lambda i: (0, 0)),
        ],
        out_specs=[
            pl.BlockSpec((NB, D), lambda i: (i, 0)),
            pl.BlockSpec((T, NB, D), lambda i: (0, i, 0)),
        ],
        out_shape=[
            jax.ShapeDtypeStruct((N, D), jnp.float32),
            jax.ShapeDtypeStruct((T, N, D), jnp.float32),
        ],
    )(x, pw, pb.reshape(1, D), ew, eb)
    return h, tr.reshape(T * N, D)


def _gru_trans_body(agg_ref, h_ref, wih_ref, whh_ref, bih_ref, bhh_ref,
                    ew_ref, eb_ref, hn_ref, tr_ref):
    hn = _gru_block(agg_ref, h_ref, wih_ref, whh_ref, bih_ref, bhh_ref)
    hn_ref[...] = hn
    _trans_block(hn, ew_ref, eb_ref, tr_ref)


def _gru_trans(agg2, h, wih, whh, bih, bhh, ew, eb):
    hn, tr = pl.pallas_call(
        _gru_trans_body,
        grid=(GRID_N,),
        in_specs=_GRU_SPECS + [
            pl.BlockSpec((T, D, D), lambda i: (0, 0, 0)),
            pl.BlockSpec((T, D), lambda i: (0, 0)),
        ],
        out_specs=[
            pl.BlockSpec((NB, D), lambda i: (i, 0)),
            pl.BlockSpec((T, NB, D), lambda i: (0, i, 0)),
        ],
        out_shape=[
            jax.ShapeDtypeStruct((N, D), jnp.float32),
            jax.ShapeDtypeStruct((T, N, D), jnp.float32),
        ],
    )(agg2, h, wih, whh, bih.reshape(1, 3 * D), bhh.reshape(1, 3 * D),
      ew, eb)
    return hn, tr.reshape(T * N, D)


def _gru_out_body(agg_ref, h_ref, wih_ref, whh_ref, bih_ref, bhh_ref,
                  ow_ref, ob_ref, o_ref):
    hn = _gru_block(agg_ref, h_ref, wih_ref, whh_ref, bih_ref, bhh_ref)
    acc = lax.dot_general(hn, ow_ref[...], (((1,), (1,)), ((), ())),
                          preferred_element_type=jnp.float32)
    o_ref[...] = acc + ob_ref[...]


def _gru_out(agg2, h, wih, whh, bih, bhh, ow, ob):
    return pl.pallas_call(
        _gru_out_body,
        grid=(GRID_N,),
        in_specs=_GRU_SPECS + [
            pl.BlockSpec((D, D), lambda i: (0, 0)),
            pl.BlockSpec((1, D), lambda i: (0, 0)),
        ],
        out_specs=pl.BlockSpec((NB, D), lambda i: (i, 0)),
        out_shape=jax.ShapeDtypeStruct((N, D), jnp.float32),
    )(agg2, h, wih, whh, bih.reshape(1, 3 * D), bhh.reshape(1, 3 * D),
      ow, ob.reshape(1, D))


def _sc_scatter(trans_flat, gidx, didx):
    mesh = plsc.VectorSubcoreMesh(core_axis_name="c", subcore_axis_name="s")

    @functools.partial(
        pl.kernel,
        out_type=jax.ShapeDtypeStruct((NC, NPAD, D), jnp.float32),
        mesh=mesh,
        scratch_types=[
            pltpu.VMEM((2, SC_CH, K), jnp.int32),
            pltpu.VMEM((2, SC_CH, K), jnp.int32),
            pltpu.VMEM((NBUF, K, D), jnp.float32),
            pltpu.VMEM_SHARED((NPAD, D), jnp.float32),
        ] + [pltpu.SemaphoreType.DMA] * (2 * NBUF + 2),
    )
    def k(trans_hbm, gidx_hbm, didx_hbm, out_hbm,
          gidx_v, didx_v, rows, agg_sh, *sems):
        gsems = sems[:NBUF]
        ssems = sems[NBUF:2 * NBUF]
        semg, semd = sems[2 * NBUF], sems[2 * NBUF + 1]
        cid = lax.axis_index("c")
        sid = lax.axis_index("s")
        wid = sid * NC + cid

        @pl.loop(0, K)
        def _(r):
            for c16 in range(D // 16):
                rows[0, r, pl.ds(c16 * 16, 16)] = jnp.zeros((16,), jnp.float32)
        for z in range(RPS // K):
            pltpu.sync_copy(rows.at[0], agg_sh.at[pl.ds(sid * RPS + z * K, K)])
        pltpu.sync_copy(rows.at[0].at[pl.ds(0, RPS % K)],
                        agg_sh.at[pl.ds(sid * RPS + (RPS // K) * K, RPS % K)])
        plsc.subcore_barrier()

        def start_gather(gv, c, b):
            pltpu.async_copy(trans_hbm.at[gv.at[c]], rows.at[b], gsems[b])

        def wait_gather(b):
            pltpu.make_async_copy(trans_hbm.at[pl.ds(0, K)], rows.at[b],
                                  gsems[b]).wait()

        def start_scatter(dv, c, b):
            pltpu.async_copy(rows.at[b], agg_sh.at[dv.at[c]], ssems[b],
                             add=True)

        def wait_scatter(b):
            pltpu.make_async_copy(rows.at[b], agg_sh.at[pl.ds(0, K)],
                                  ssems[b]).wait()

        pltpu.sync_copy(gidx_hbm.at[wid].at[0], gidx_v.at[0])
        pltpu.sync_copy(didx_hbm.at[wid].at[0], didx_v.at[0])
        idx_cp = None
        for sb in range(SB):
            ib = sb % 2
            if idx_cp is not None:
                for cp in idx_cp:
                    cp.wait()
            if sb + 1 < SB:
                idx_cp = (
                    pltpu.async_copy(gidx_hbm.at[wid].at[sb + 1],
                                     gidx_v.at[1 - ib], semg),
                    pltpu.async_copy(didx_hbm.at[wid].at[sb + 1],
                                     didx_v.at[1 - ib], semd),
                )
            gv, dv = gidx_v.at[ib], didx_v.at[ib]
            for b in range(NBUF):
                start_gather(gv, b, b)

            @pl.loop(1, NG)
            def _(i):
                c = NBUF * i
                for b in range(NBUF):
                    wait_gather(b)
                    start_scatter(dv, c - NBUF + b, b)
                for b in range(NBUF):
                    wait_scatter(b)
                    start_gather(gv, c + b, b)

            for b in range(NBUF):
                wait_gather(b)
                start_scatter(dv, (NG - 1) * NBUF + b, b)
            for t in range(NTAIL):
                wait_scatter(t)
                start_gather(gv, NG * NBUF + t, t)
            for t in range(NTAIL):
                wait_gather(t)
                start_scatter(dv, NG * NBUF + t, t)
            for b in range(NBUF):
                wait_scatter(b)

        plsc.subcore_barrier()
        pltpu.sync_copy(agg_sh.at[pl.ds(sid * RPS, RPS)],
                        out_hbm.at[cid].at[pl.ds(sid * RPS, RPS)])

    return k(trans_flat, gidx, didx)


def kernel(node_features, edge_index, edge_type, proj_W, proj_b,
           edge_W0, edge_b0, gru_Wih0, gru_Whh0, gru_bih0, gru_bhh0,
           edge_W1, edge_b1, gru_Wih1, gru_Whh1, gru_bih1, gru_bhh1,
           out_W, out_b):
    src = edge_index[0].astype(jnp.int32)
    dst = edge_index[1].astype(jnp.int32)
    gidx = (edge_type.astype(jnp.int32) * N + src).reshape(NW, SB, SC_CH, K)
    didx = dst.reshape(NW, SB, SC_CH, K)

    layers = [
        (edge_W0, edge_b0, gru_Wih0, gru_Whh0, gru_bih0, gru_bhh0),
        (edge_W1, edge_b1, gru_Wih1, gru_Whh1, gru_bih1, gru_bhh1),
    ]
    steps = [layers[0], layers[1], layers[0], layers[1]]

    h, trans = _proj_trans(node_features, proj_W, proj_b,
                           steps[0][0], steps[0][1])
    for s in range(4):
        (_eW, _eb, Wih, Whh, bih, bhh) = steps[s]
        agg2 = _sc_scatter(trans, gidx, didx)
        if s < 3:
            n_eW, n_eb = steps[s + 1][0], steps[s + 1][1]
            h, trans = _gru_trans(agg2, h, Wih, Whh, bih, bhh, n_eW, n_eb)
        else:
            out = _gru_out(agg2, h, Wih, Whh, bih, bhh, out_W, out_b)
    return out
